# baseline (speedup 1.0000x reference)
import jax
import jax.numpy as jnp
from jax.experimental import pallas as pl
from jax.experimental.pallas import tpu as pltpu

B, D, H, L, K = 4096, 768, 1024, 256, 1024
TB = 1024
GRID = B // TB


_SQRT_HALF = 0.7071067811865476


def _gelu(v):
    return 0.5 * v * (1.0 + jax.lax.erf(v * _SQRT_HALF))


def _fused_kernel(x_ref, embed_hbm,
                  ew1_ref, eb1_ref, ew2_hbm, eb2_ref, ew3_hbm, eb3_ref,
                  dw1_hbm, db1_ref, dw2_hbm, db2_ref, dw3_hbm, db3_ref,
                  xp_ref, zd_ref, loss_ref,
                  ew2_v, ew3_v, emb_v, dw1_v, dw2_v, dw3_v, sems):
    i = pl.program_id(0)

    c_ew2 = pltpu.make_async_copy(ew2_hbm, ew2_v, sems.at[0])
    c_ew3 = pltpu.make_async_copy(ew3_hbm, ew3_v, sems.at[1])
    c_emb = pltpu.make_async_copy(embed_hbm, emb_v, sems.at[2])
    c_dw1 = pltpu.make_async_copy(dw1_hbm, dw1_v, sems.at[3])
    c_dw2 = pltpu.make_async_copy(dw2_hbm, dw2_v, sems.at[4])
    c_dw3 = pltpu.make_async_copy(dw3_hbm, dw3_v, sems.at[5])

    @pl.when(i == 0)
    def _start_streams():
        c_ew2.start()
        c_ew3.start()
        c_emb.start()
        c_dw1.start()
        c_dw2.start()
        c_dw3.start()

    x = x_ref[...]

    h = _gelu(jnp.dot(x, ew1_ref[...], preferred_element_type=jnp.float32)
              + eb1_ref[...])

    @pl.when(i == 0)
    def _wait_ew2():
        c_ew2.wait()

    h = _gelu(jnp.dot(h, ew2_v[...], preferred_element_type=jnp.float32)
              + eb2_ref[...])

    @pl.when(i == 0)
    def _wait_ew3():
        c_ew3.wait()
        c_emb.wait()

    z_e = (jnp.dot(h, ew3_v[...], preferred_element_type=jnp.float32)
           + eb3_ref[...])

    embed = emb_v[...]
    e2 = jnp.sum(embed * embed, axis=1)[None, :]
    ze2 = jnp.sum(z_e * z_e, axis=1, keepdims=True)
    cross = jax.lax.dot_general(
        z_e, embed, (((1,), (1,)), ((), ())),
        preferred_element_type=jnp.float32)
    d2 = ze2 + e2 - 2.0 * cross

    m = jnp.min(d2, axis=1, keepdims=True)
    iota = jax.lax.broadcasted_iota(jnp.int32, (TB, K), 1)
    idx = jnp.min(jnp.where(d2 == m, iota, K), axis=1, keepdims=True)
    onehot = (iota == idx).astype(jnp.float32)
    zd_ref[...] = onehot.astype(jnp.int32)

    z_q = jnp.dot(onehot, embed, preferred_element_type=jnp.float32)

    @pl.when(i == 0)
    def _wait_dec():
        c_dw1.wait()
        c_dw2.wait()
        c_dw3.wait()

    g = _gelu(jnp.dot(z_q, dw1_v[...], preferred_element_type=jnp.float32)
              + db1_ref[...])
    g = _gelu(jnp.dot(g, dw2_v[...], preferred_element_type=jnp.float32)
              + db2_ref[...])
    x_pred = jax.nn.sigmoid(
        jnp.dot(g, dw3_v[...], preferred_element_type=jnp.float32)
        + db3_ref[...])
    xp_ref[...] = x_pred

    dxe = x - x_pred
    dzq = z_e - z_q
    partial = ((jnp.sum(dxe * dxe) + 1.25 * jnp.sum(dzq * dzq))
               * (1.0 / B)).reshape(1, 1)

    @pl.when(i == 0)
    def _init():
        loss_ref[...] = partial

    @pl.when(i != 0)
    def _acc():
        loss_ref[...] += partial


def _full(shape):
    return pl.BlockSpec(shape, lambda i: tuple(0 for _ in shape))


def _hbm():
    return pl.BlockSpec(memory_space=pl.ANY)


@jax.jit
def kernel(x, embed, enc_w1, enc_b1, enc_w2, enc_b2, enc_w3, enc_b3,
           dec_w1, dec_b1, dec_w2, dec_b2, dec_w3, dec_b3):
    biases = [b.reshape(1, -1) for b in
              (enc_b1, enc_b2, enc_b3, dec_b1, dec_b2, dec_b3)]
    eb1, eb2, eb3, db1, db2, db3 = biases

    in_specs = [
            pl.BlockSpec((TB, D), lambda i: (i, 0)),
            _hbm(),
            _full((D, H)), _full((1, H)),
            _hbm(), _full((1, H)),
            _hbm(), _full((1, L)),
            _hbm(), _full((1, H)),
            _hbm(), _full((1, H)),
            _hbm(), _full((1, D)),
    ]
    out_specs = [
        pl.BlockSpec((TB, D), lambda i: (i, 0)),
        pl.BlockSpec((TB, K), lambda i: (i, 0)),
        pl.BlockSpec((1, 1), lambda i: (0, 0)),
    ]
    x_pred, z_disc, loss = pl.pallas_call(
        _fused_kernel,
        grid=(GRID,),
        in_specs=in_specs,
        out_specs=out_specs,
        out_shape=[
            jax.ShapeDtypeStruct((B, D), jnp.float32),
            jax.ShapeDtypeStruct((B, K), jnp.int32),
            jax.ShapeDtypeStruct((1, 1), jnp.float32),
        ],
        scratch_shapes=[
            pltpu.VMEM((H, H), jnp.float32),
            pltpu.VMEM((H, L), jnp.float32),
            pltpu.VMEM((K, L), jnp.float32),
            pltpu.VMEM((L, H), jnp.float32),
            pltpu.VMEM((H, H), jnp.float32),
            pltpu.VMEM((H, D), jnp.float32),
            pltpu.SemaphoreType.DMA((6,)),
        ],
        compiler_params=pltpu.CompilerParams(
            dimension_semantics=("arbitrary",),
        ),
    )(x, embed, enc_w1, eb1, enc_w2, eb2, enc_w3, eb3,
      dec_w1, db1, dec_w2, db2, dec_w3, db3)
    return (x_pred, z_disc, loss[0, 0])


# speedup vs baseline: 1.0200x; 1.0200x over previous
import jax
import jax.numpy as jnp
from jax.experimental import pallas as pl
from jax.experimental.pallas import tpu as pltpu

B, D, H, L, K = 4096, 768, 1024, 256, 1024
TB = 1024
GRID = B // TB


_SQRT_HALF = 0.7071067811865476


def _gelu(v):
    return 0.5 * v * (1.0 + jax.lax.erf(v * _SQRT_HALF))


def _fused_kernel(x_ref, embed_ref,
                  ew1_ref, eb1_ref, ew2_ref, eb2_ref, ew3_ref, eb3_ref,
                  dw1_ref, db1_ref, dw2_hbm, db2_ref, dw3_hbm, db3_ref,
                  xp_ref, zd_ref, loss_ref,
                  dw2_v, dw3_v, sems):
    i = pl.program_id(0)

    c_dw2 = pltpu.make_async_copy(dw2_hbm, dw2_v, sems.at[0])
    c_dw3 = pltpu.make_async_copy(dw3_hbm, dw3_v, sems.at[1])

    @pl.when(i == 0)
    def _start_streams():
        c_dw2.start()
        c_dw3.start()

    x = x_ref[...]

    h = _gelu(jnp.dot(x, ew1_ref[...], preferred_element_type=jnp.float32)
              + eb1_ref[...])
    h = _gelu(jnp.dot(h, ew2_ref[...], preferred_element_type=jnp.float32)
              + eb2_ref[...])
    z_e = (jnp.dot(h, ew3_ref[...], preferred_element_type=jnp.float32)
           + eb3_ref[...])

    embed = embed_ref[...]
    e2 = jnp.sum(embed * embed, axis=1)[None, :]
    ze2 = jnp.sum(z_e * z_e, axis=1, keepdims=True)
    cross = jax.lax.dot_general(
        z_e, embed, (((1,), (1,)), ((), ())),
        preferred_element_type=jnp.float32)
    d2 = ze2 + e2 - 2.0 * cross

    m = jnp.min(d2, axis=1, keepdims=True)
    iota = jax.lax.broadcasted_iota(jnp.int32, (TB, K), 1)
    idx = jnp.min(jnp.where(d2 == m, iota, K), axis=1, keepdims=True)
    onehot = (iota == idx).astype(jnp.float32)
    zd_ref[...] = onehot.astype(jnp.int32)

    z_q = jnp.dot(onehot, embed, preferred_element_type=jnp.float32)

    g = _gelu(jnp.dot(z_q, dw1_ref[...], preferred_element_type=jnp.float32)
              + db1_ref[...])

    @pl.when(i == 0)
    def _wait_dec():
        c_dw2.wait()
        c_dw3.wait()

    g = _gelu(jnp.dot(g, dw2_v[...], preferred_element_type=jnp.float32)
              + db2_ref[...])
    x_pred = jax.nn.sigmoid(
        jnp.dot(g, dw3_v[...], preferred_element_type=jnp.float32)
        + db3_ref[...])
    xp_ref[...] = x_pred

    dxe = x - x_pred
    dzq = z_e - z_q
    partial = ((jnp.sum(dxe * dxe) + 1.25 * jnp.sum(dzq * dzq))
               * (1.0 / B)).reshape(1, 1)

    @pl.when(i == 0)
    def _init():
        loss_ref[...] = partial

    @pl.when(i != 0)
    def _acc():
        loss_ref[...] += partial


def _full(shape):
    return pl.BlockSpec(shape, lambda i: tuple(0 for _ in shape))


def _hbm():
    return pl.BlockSpec(memory_space=pl.ANY)


@jax.jit
def kernel(x, embed, enc_w1, enc_b1, enc_w2, enc_b2, enc_w3, enc_b3,
           dec_w1, dec_b1, dec_w2, dec_b2, dec_w3, dec_b3):
    biases = [b.reshape(1, -1) for b in
              (enc_b1, enc_b2, enc_b3, dec_b1, dec_b2, dec_b3)]
    eb1, eb2, eb3, db1, db2, db3 = biases

    in_specs = [
        pl.BlockSpec((TB, D), lambda i: (i, 0)),
        _full((K, L)),
        _full((D, H)), _full((1, H)),
        _full((H, H)), _full((1, H)),
        _full((H, L)), _full((1, L)),
        _full((L, H)), _full((1, H)),
        _hbm(), _full((1, H)),
        _hbm(), _full((1, D)),
    ]
    out_specs = [
        pl.BlockSpec((TB, D), lambda i: (i, 0)),
        pl.BlockSpec((TB, K), lambda i: (i, 0)),
        pl.BlockSpec((1, 1), lambda i: (0, 0)),
    ]
    x_pred, z_disc, loss = pl.pallas_call(
        _fused_kernel,
        grid=(GRID,),
        in_specs=in_specs,
        out_specs=out_specs,
        out_shape=[
            jax.ShapeDtypeStruct((B, D), jnp.float32),
            jax.ShapeDtypeStruct((B, K), jnp.int32),
            jax.ShapeDtypeStruct((1, 1), jnp.float32),
        ],
        scratch_shapes=[
            pltpu.VMEM((H, H), jnp.float32),
            pltpu.VMEM((H, D), jnp.float32),
            pltpu.SemaphoreType.DMA((2,)),
        ],
        compiler_params=pltpu.CompilerParams(
            dimension_semantics=("arbitrary",),
        ),
    )(x, embed, enc_w1, eb1, enc_w2, eb2, enc_w3, eb3,
      dec_w1, db1, dec_w2, db2, dec_w3, db3)
    return (x_pred, z_disc, loss[0, 0])


# speedup vs baseline: 1.0514x; 1.0308x over previous
import jax
import jax.numpy as jnp
from jax.experimental import pallas as pl
from jax.experimental.pallas import tpu as pltpu

B, D, H, L, K = 4096, 768, 1024, 256, 1024
TB = 1024
GRID = B // TB


_SQRT_HALF = 0.7071067811865476


def _gelu(v):
    return 0.5 * v * (1.0 + jax.lax.erf(v * _SQRT_HALF))


def _fused_kernel(x_ref, embed_ref,
                  ew1_ref, eb1_ref, ew2_ref, eb2_ref, ew3_ref, eb3_ref,
                  dw1_ref, db1_ref, dw2_ref, db2_ref, dw3_ref, db3_ref,
                  xp_ref, zd_ref, loss_ref):
    x = x_ref[...]

    h = _gelu(jnp.dot(x, ew1_ref[...], preferred_element_type=jnp.float32)
              + eb1_ref[...])
    h = _gelu(jnp.dot(h, ew2_ref[...], preferred_element_type=jnp.float32)
              + eb2_ref[...])
    z_e = (jnp.dot(h, ew3_ref[...], preferred_element_type=jnp.float32)
           + eb3_ref[...])

    embed = embed_ref[...]
    e2 = jnp.sum(embed * embed, axis=1)[None, :]
    ze2 = jnp.sum(z_e * z_e, axis=1, keepdims=True)
    cross = jax.lax.dot_general(
        z_e, embed, (((1,), (1,)), ((), ())),
        preferred_element_type=jnp.float32)
    d2 = ze2 + e2 - 2.0 * cross

    m = jnp.min(d2, axis=1, keepdims=True)
    iota = jax.lax.broadcasted_iota(jnp.int32, (TB, K), 1)
    idx = jnp.min(jnp.where(d2 == m, iota, K), axis=1, keepdims=True)
    onehot = (iota == idx).astype(jnp.float32)
    zd_ref[...] = onehot.astype(jnp.int32)

    z_q = jnp.dot(onehot, embed, preferred_element_type=jnp.float32)

    g = _gelu(jnp.dot(z_q, dw1_ref[...], preferred_element_type=jnp.float32)
              + db1_ref[...])
    g = _gelu(jnp.dot(g, dw2_ref[...], preferred_element_type=jnp.float32)
              + db2_ref[...])
    x_pred = jax.nn.sigmoid(
        jnp.dot(g, dw3_ref[...], preferred_element_type=jnp.float32)
        + db3_ref[...])
    xp_ref[...] = x_pred

    dxe = x - x_pred
    dzq = z_e - z_q
    partial = ((jnp.sum(dxe * dxe) + 1.25 * jnp.sum(dzq * dzq))
               * (1.0 / B)).reshape(1, 1)
    loss_ref[...] = jnp.broadcast_to(partial, (1, 128)).reshape(1, 1, 128)


def _full(shape):
    return pl.BlockSpec(shape, lambda i: tuple(0 for _ in shape))


@jax.jit
def kernel(x, embed, enc_w1, enc_b1, enc_w2, enc_b2, enc_w3, enc_b3,
           dec_w1, dec_b1, dec_w2, dec_b2, dec_w3, dec_b3):
    biases = [b.reshape(1, -1) for b in
              (enc_b1, enc_b2, enc_b3, dec_b1, dec_b2, dec_b3)]
    eb1, eb2, eb3, db1, db2, db3 = biases

    in_specs = [
        pl.BlockSpec((TB, D), lambda i: (i, 0)),
        _full((K, L)),
        _full((D, H)), _full((1, H)),
        _full((H, H)), _full((1, H)),
        _full((H, L)), _full((1, L)),
        _full((L, H)), _full((1, H)),
        _full((H, H)), _full((1, H)),
        _full((H, D)), _full((1, D)),
    ]
    out_specs = [
        pl.BlockSpec((TB, D), lambda i: (i, 0)),
        pl.BlockSpec((TB, K), lambda i: (i, 0)),
        pl.BlockSpec((1, 1, 128), lambda i: (i, 0, 0)),
    ]
    x_pred, z_disc, partials = pl.pallas_call(
        _fused_kernel,
        grid=(GRID,),
        in_specs=in_specs,
        out_specs=out_specs,
        out_shape=[
            jax.ShapeDtypeStruct((B, D), jnp.float32),
            jax.ShapeDtypeStruct((B, K), jnp.int32),
            jax.ShapeDtypeStruct((GRID, 1, 128), jnp.float32),
        ],
        compiler_params=pltpu.CompilerParams(
            dimension_semantics=("arbitrary",),
        ),
    )(x, embed, enc_w1, eb1, enc_w2, eb2, enc_w3, eb3,
      dec_w1, db1, dec_w2, db2, dec_w3, db3)
    return (x_pred, z_disc, jnp.sum(partials[:, 0, 0]))


# speedup vs baseline: 1.0846x; 1.0315x over previous
import jax
import jax.numpy as jnp
from jax.experimental import pallas as pl
from jax.experimental.pallas import tpu as pltpu

B, D, H, L, K = 4096, 768, 1024, 256, 1024
TB = 1024
CH = 512
NC = TB // CH
GRID = B // TB

_SQRT_HALF = 0.7071067811865476


def _gelu(v):
    return 0.5 * v * (1.0 + jax.lax.erf(v * _SQRT_HALF))


def _fused_kernel(x_ref, embed_ref,
                  ew1_ref, eb1_ref, ew2_ref, eb2_ref, ew3_ref, eb3_ref,
                  dw1_ref, db1_ref, dw2_ref, db2_ref, dw3_ref, db3_ref,
                  xp_ref, zd_ref, loss_ref):
    embed = embed_ref[...]
    e2 = jnp.sum(embed * embed, axis=1)[None, :]

    st = [dict() for _ in range(NC)]

    def s0(c):
        x = x_ref[pl.ds(c * CH, CH), :]
        st[c]["x"] = x
        st[c]["h"] = _gelu(
            jnp.dot(x, ew1_ref[...], preferred_element_type=jnp.float32)
            + eb1_ref[...])

    def s1(c):
        st[c]["h"] = _gelu(
            jnp.dot(st[c]["h"], ew2_ref[...],
                    preferred_element_type=jnp.float32)
            + eb2_ref[...])

    def s2(c):
        z_e = (jnp.dot(st[c]["h"], ew3_ref[...],
                       preferred_element_type=jnp.float32)
               + eb3_ref[...])
        st[c]["z_e"] = z_e
        st[c]["ze2"] = jnp.sum(z_e * z_e, axis=1, keepdims=True)

    def s3(c):
        cross = jax.lax.dot_general(
            st[c]["z_e"], embed, (((1,), (1,)), ((), ())),
            preferred_element_type=jnp.float32)
        st[c]["d2"] = st[c]["ze2"] + e2 - 2.0 * cross

    def s4(c):
        d2 = st[c]["d2"]
        m = jnp.min(d2, axis=1, keepdims=True)
        iota = jax.lax.broadcasted_iota(jnp.int32, (CH, K), 1)
        idx = jnp.min(jnp.where(d2 == m, iota, K), axis=1, keepdims=True)
        onehot = (iota == idx).astype(jnp.float32)
        st[c]["onehot"] = onehot
        zd_ref[pl.ds(c * CH, CH), :] = onehot.astype(jnp.int32)

    def s5(c):
        z_q = jnp.dot(st[c]["onehot"], embed,
                      preferred_element_type=jnp.float32)
        st[c]["z_q"] = z_q
        st[c]["g"] = _gelu(
            jnp.dot(z_q, dw1_ref[...], preferred_element_type=jnp.float32)
            + db1_ref[...])

    def s6(c):
        st[c]["g"] = _gelu(
            jnp.dot(st[c]["g"], dw2_ref[...],
                    preferred_element_type=jnp.float32)
            + db2_ref[...])

    def s7(c):
        x_pred = jax.nn.sigmoid(
            jnp.dot(st[c]["g"], dw3_ref[...],
                    preferred_element_type=jnp.float32)
            + db3_ref[...])
        xp_ref[pl.ds(c * CH, CH), :] = x_pred
        dxe = st[c]["x"] - x_pred
        dzq = st[c]["z_e"] - st[c]["z_q"]
        st[c]["partial"] = ((jnp.sum(dxe * dxe)
                             + 1.25 * jnp.sum(dzq * dzq)) * (1.0 / B))

    stages = [s0, s1, s2, s3, s4, s5, s6, s7]
    for t in range(len(stages) + NC - 1):
        for c in range(NC):
            s = t - c
            if 0 <= s < len(stages):
                stages[s](c)

    partial = sum(st[c]["partial"] for c in range(NC)).reshape(1, 1)
    loss_ref[...] = jnp.broadcast_to(partial, (1, 128)).reshape(1, 1, 128)


def _full(shape):
    return pl.BlockSpec(shape, lambda i: tuple(0 for _ in shape))


@jax.jit
def kernel(x, embed, enc_w1, enc_b1, enc_w2, enc_b2, enc_w3, enc_b3,
           dec_w1, dec_b1, dec_w2, dec_b2, dec_w3, dec_b3):
    biases = [b.reshape(1, -1) for b in
              (enc_b1, enc_b2, enc_b3, dec_b1, dec_b2, dec_b3)]
    eb1, eb2, eb3, db1, db2, db3 = biases

    in_specs = [
        pl.BlockSpec((TB, D), lambda i: (i, 0)),
        _full((K, L)),
        _full((D, H)), _full((1, H)),
        _full((H, H)), _full((1, H)),
        _full((H, L)), _full((1, L)),
        _full((L, H)), _full((1, H)),
        _full((H, H)), _full((1, H)),
        _full((H, D)), _full((1, D)),
    ]
    out_specs = [
        pl.BlockSpec((TB, D), lambda i: (i, 0)),
        pl.BlockSpec((TB, K), lambda i: (i, 0)),
        pl.BlockSpec((1, 1, 128), lambda i: (i, 0, 0)),
    ]
    x_pred, z_disc, partials = pl.pallas_call(
        _fused_kernel,
        grid=(GRID,),
        in_specs=in_specs,
        out_specs=out_specs,
        out_shape=[
            jax.ShapeDtypeStruct((B, D), jnp.float32),
            jax.ShapeDtypeStruct((B, K), jnp.int32),
            jax.ShapeDtypeStruct((GRID, 1, 128), jnp.float32),
        ],
        compiler_params=pltpu.CompilerParams(
            dimension_semantics=("arbitrary",),
        ),
    )(x, embed, enc_w1, eb1, enc_w2, eb2, enc_w3, eb3,
      dec_w1, db1, dec_w2, db2, dec_w3, db3)
    return (x_pred, z_disc, jnp.sum(partials[:, 0, 0]))
